# baseline (speedup 1.0000x reference)
import functools

import numpy as np
import jax
import jax.numpy as jnp
from jax import lax
from jax.experimental import pallas as pl
from jax.experimental.pallas import tpu as pltpu
from jax.experimental.pallas import tpu_sc as plsc

N_TOK = 5
N_CLIENTS = 1000
D = 128
B = 1024
S = 200
MAIN = S - N_TOK
PREF_ROWS = N_CLIENTS * N_TOK

NC = 2
NS = 16
NW = NC * NS
B_PER_W = B // NW

_NOISE = np.asarray(
    jax.random.normal(jax.random.key(42), (N_CLIENTS, N_TOK, D),
                      dtype=jnp.float32))


def _sc_gather(tokens_flat, idx_pref, wte, samp):
    mesh = plsc.VectorSubcoreMesh(core_axis_name="c", subcore_axis_name="s")

    @functools.partial(
        pl.kernel,
        out_type=jax.ShapeDtypeStruct((B, S, D), jnp.float32),
        mesh=mesh,
        scratch_types=[
            pltpu.VMEM((B_PER_W * S,), jnp.int32),
            pltpu.VMEM((B_PER_W * 8,), jnp.int32),
            pltpu.VMEM((S + 8, D), jnp.float32),
            pltpu.VMEM((S + 8, D), jnp.float32),
            pltpu.VMEM((S + 8, D), jnp.float32),
            pltpu.VMEM((S + 8, D), jnp.float32),
            pltpu.SemaphoreType.DMA,
            pltpu.SemaphoreType.DMA,
            pltpu.SemaphoreType.DMA,
            pltpu.SemaphoreType.DMA,
            pltpu.SemaphoreType.DMA,
            pltpu.SemaphoreType.DMA,
            pltpu.SemaphoreType.DMA,
            pltpu.SemaphoreType.DMA,
        ],
    )
    def k(tok_hbm, idx_pref_hbm, wte_hbm, samp_hbm, out_hbm,
          idx_m_v, idx_p_v, blk0, blk1, blk2, blk3,
          sg0, sg1, sg2, sg3, sw0, sw1, sw2, sw3):
        wid = lax.axis_index("s") * NC + lax.axis_index("c")
        base = wid * B_PER_W
        blk = (blk0, blk1, blk2, blk3)
        sg = (sg0, sg1, sg2, sg3)
        sw = (sw0, sw1, sw2, sw3)

        pltpu.sync_copy(tok_hbm.at[pl.ds(base * S, B_PER_W * S)], idx_m_v)
        pltpu.sync_copy(idx_pref_hbm.at[pl.ds(base * 8, B_PER_W * 8)], idx_p_v)

        def start_gather(i, s):
            pltpu.async_copy(
                wte_hbm.at[idx_m_v.at[pl.ds(i * S, S)]],
                blk[s].at[pl.ds(0, S)], sg[s])
            pltpu.async_copy(
                samp_hbm.at[idx_p_v.at[pl.ds(i * 8, N_TOK)]],
                blk[s].at[pl.ds(S, N_TOK)], sg[s])

        def wait_gather(i, s):
            pltpu.make_async_copy(out_hbm.at[base + i],
                                  blk[s].at[pl.ds(0, S)], sg[s]).wait()
            pltpu.make_async_copy(samp_hbm.at[pl.ds(0, N_TOK)],
                                  blk[s].at[pl.ds(S, N_TOK)], sg[s]).wait()

        def start_write(i, s):
            pltpu.async_copy(blk[s].at[pl.ds(N_TOK, S)],
                             out_hbm.at[base + i], sw[s])

        def wait_write(i, s):
            pltpu.make_async_copy(blk[s].at[pl.ds(N_TOK, S)],
                                  out_hbm.at[base + i], sw[s]).wait()

        start_gather(0, 0)

        def group_body(g, _):
            for s in range(4):
                i = 4 * g + s
                ns = (s + 1) % 4

                @pl.when(i >= 3)
                def _():
                    wait_write(i - 3, ns)

                @pl.when(i + 1 < B_PER_W)
                def _():
                    start_gather(i + 1, ns)

                wait_gather(i, s)
                start_write(i, s)
            return ()

        lax.fori_loop(0, B_PER_W // 4, group_body, ())
        wait_write(B_PER_W - 3, 1)
        wait_write(B_PER_W - 2, 2)
        wait_write(B_PER_W - 1, 3)

    return k(tokens_flat, idx_pref, wte, samp)


@jax.jit
def kernel(tokens, wte_weight, avgs, vars_):
    samp = (avgs + vars_ * jnp.asarray(_NOISE)).reshape(PREF_ROWS, D)

    cid = tokens[:, 0]
    pbase = ((cid + N_CLIENTS - 1) % N_CLIENTS) * N_TOK
    offs = jnp.array([0, 1, 2, 3, 4, 0, 0, 0], jnp.int32)
    idx_pref = pbase[:, None] + offs[None, :]

    return _sc_gather(tokens.reshape(-1), idx_pref.reshape(-1),
                      wte_weight, samp)


# speedup vs baseline: 1.0088x; 1.0088x over previous
import functools

import numpy as np
import jax
import jax.numpy as jnp
from jax import lax
from jax.experimental import pallas as pl
from jax.experimental.pallas import tpu as pltpu
from jax.experimental.pallas import tpu_sc as plsc

N_TOK = 5
N_CLIENTS = 1000
D = 128
B = 1024
S = 200
MAIN = S - N_TOK
PREF_ROWS = N_CLIENTS * N_TOK

NC = 2
NS = 16
NW = NC * NS
B_PER_W = B // NW

_NOISE = np.asarray(
    jax.random.normal(jax.random.key(42), (N_CLIENTS, N_TOK, D),
                      dtype=jnp.float32))


def _sc_gather(tokens_flat, idx_pref, wte, samp):
    mesh = plsc.VectorSubcoreMesh(core_axis_name="c", subcore_axis_name="s")

    @functools.partial(
        pl.kernel,
        out_type=jax.ShapeDtypeStruct((B, S, D), jnp.float32),
        mesh=mesh,
        scratch_types=[
            pltpu.VMEM((B_PER_W * S,), jnp.int32),
            pltpu.VMEM((B_PER_W * 8,), jnp.int32),
            pltpu.VMEM((S + 8, D), jnp.float32),
            pltpu.VMEM((S + 8, D), jnp.float32),
            pltpu.SemaphoreType.DMA,
            pltpu.SemaphoreType.DMA,
            pltpu.SemaphoreType.DMA,
            pltpu.SemaphoreType.DMA,
        ],
    )
    def k(tok_hbm, idx_pref_hbm, wte_hbm, samp_hbm, out_hbm,
          idx_m_v, idx_p_v, blk0, blk1,
          sg0, sg1, sw0, sw1):
        wid = lax.axis_index("s") * NC + lax.axis_index("c")
        base = wid * B_PER_W
        blk = (blk0, blk1)
        sg = (sg0, sg1)
        sw = (sw0, sw1)

        pltpu.sync_copy(tok_hbm.at[pl.ds(base * S, B_PER_W * S)], idx_m_v)
        pltpu.sync_copy(idx_pref_hbm.at[pl.ds(base * 8, B_PER_W * 8)], idx_p_v)

        def start_gather(i, s):
            pltpu.async_copy(
                wte_hbm.at[idx_m_v.at[pl.ds(i * S, S)]],
                blk[s].at[pl.ds(0, S)], sg[s])
            pltpu.async_copy(
                samp_hbm.at[idx_p_v.at[pl.ds(i * 8, N_TOK)]],
                blk[s].at[pl.ds(S, N_TOK)], sg[s])

        def wait_gather(i, s):
            pltpu.make_async_copy(out_hbm.at[base + i],
                                  blk[s].at[pl.ds(0, S)], sg[s]).wait()
            pltpu.make_async_copy(samp_hbm.at[pl.ds(0, N_TOK)],
                                  blk[s].at[pl.ds(S, N_TOK)], sg[s]).wait()

        def start_write(i, s):
            pltpu.async_copy(blk[s].at[pl.ds(N_TOK, S)],
                             out_hbm.at[base + i], sw[s])

        def wait_write(i, s):
            pltpu.make_async_copy(blk[s].at[pl.ds(N_TOK, S)],
                                  out_hbm.at[base + i], sw[s]).wait()

        start_gather(0, 0)
        npair = B_PER_W // 2

        def pair_body(p, _):
            i0 = 2 * p
            i1 = i0 + 1

            @pl.when(p >= 1)
            def _():
                wait_write(i0 - 1, 1)

            start_gather(i1, 1)
            wait_gather(i0, 0)
            start_write(i0, 0)

            @pl.when(p + 1 < npair)
            def _():
                wait_write(i0, 0)
                start_gather(i0 + 2, 0)

            wait_gather(i1, 1)
            start_write(i1, 1)
            return ()

        lax.fori_loop(0, npair, pair_body, ())
        wait_write(B_PER_W - 2, 0)
        wait_write(B_PER_W - 1, 1)

    return k(tokens_flat, idx_pref, wte, samp)


@jax.jit
def kernel(tokens, wte_weight, avgs, vars_):
    samp = (avgs + vars_ * jnp.asarray(_NOISE)).reshape(PREF_ROWS, D)

    cid = tokens[:, 0]
    pbase = ((cid + N_CLIENTS - 1) % N_CLIENTS) * N_TOK
    offs = jnp.array([0, 1, 2, 3, 4, 0, 0, 0], jnp.int32)
    idx_pref = pbase[:, None] + offs[None, :]

    return _sc_gather(tokens.reshape(-1), idx_pref.reshape(-1),
                      wte_weight, samp)
